# Initial kernel scaffold; baseline (speedup 1.0000x reference)
import jax
import jax.numpy as jnp
from jax.experimental import pallas as pl


def kernel(node_feats, edge_feats, edge_index, Qw, Kw, Vw, Ew):
    raise NotImplementedError("write your pallas kernel here")


# speedup vs baseline: 8.0456x; 8.0456x over previous
import jax
import jax.numpy as jnp
from jax import lax
from jax.experimental import pallas as pl
from jax.experimental.pallas import tpu as pltpu

N = 10000
E = 320000
DIN = 128
H = 8
D = 16
HD = H * D
EB = 512
NB = E // EB


def _tab_body(x_ref, wk_ref, wq_ref, wv_ref, tk_ref, tq_ref, tv_ref):
    x = x_ref[...]
    tk_ref[...] = jnp.dot(x, wk_ref[...], preferred_element_type=jnp.float32)
    tq_ref[...] = jnp.dot(x, wq_ref[...], preferred_element_type=jnp.float32)
    tv_ref[...] = jnp.dot(x, wv_ref[...], preferred_element_type=jnp.float32)


def _tab_call(node_feats, wk, wq, wv):
    bn = 2000
    return pl.pallas_call(
        _tab_body,
        grid=(N // bn,),
        in_specs=[
            pl.BlockSpec((bn, DIN), lambda i: (i, 0)),
            pl.BlockSpec((DIN, HD), lambda i: (0, 0)),
            pl.BlockSpec((DIN, HD), lambda i: (0, 0)),
            pl.BlockSpec((DIN, HD), lambda i: (0, 0)),
        ],
        out_specs=[
            pl.BlockSpec((bn, HD), lambda i: (i, 0)),
            pl.BlockSpec((bn, HD), lambda i: (i, 0)),
            pl.BlockSpec((bn, HD), lambda i: (i, 0)),
        ],
        out_shape=[jax.ShapeDtypeStruct((N, HD), jnp.float32)] * 3,
    )(node_feats, wk, wq, wv)


def _edge_body(src_ref, dst_ref, x_ref, we_ref, tk_ref, tq_ref, tv_ref,
               eout_ref, wv_ref, z_ref, vsrc_ref, pe_ref, sblk_ref, msg_ref):
    i = pl.program_id(0)

    @pl.when(i == 0)
    def _():
        wv_ref[...] = jnp.zeros((N, HD), jnp.float32)
        z_ref[...] = jnp.zeros((N, HD), jnp.float32)

    pe_ref[...] = jnp.dot(x_ref[...], we_ref[...],
                          preferred_element_type=jnp.float32)

    def gather(e, carry):
        s = src_ref[e]
        d = dst_ref[e]
        kq = tk_ref[pl.ds(s, 1), :] * tq_ref[pl.ds(d, 1), :]
        sc = jnp.minimum(jnp.maximum(kq, -5.0), 5.0) * pe_ref[pl.ds(e, 1), :]
        eout_ref[pl.ds(e, 1), :] = sc
        vsrc_ref[pl.ds(e, 1), :] = tv_ref[pl.ds(s, 1), :]
        return carry

    lax.fori_loop(0, EB, gather, 0)

    r = lax.broadcasted_iota(jnp.int32, (HD, HD), 0)
    c = lax.broadcasted_iota(jnp.int32, (HD, HD), 1)
    summat = jnp.where(r // D == c // D, 1.0, 0.0)
    logits = jnp.dot(eout_ref[...], summat,
                     preferred_element_type=jnp.float32)
    sblk_ref[...] = jnp.exp(jnp.minimum(jnp.maximum(logits, -5.0), 5.0))
    msg_ref[...] = vsrc_ref[...] * sblk_ref[...]

    def scatter(e, carry):
        d = dst_ref[e]
        wv_ref[pl.ds(d, 1), :] += msg_ref[pl.ds(e, 1), :]
        z_ref[pl.ds(d, 1), :] += sblk_ref[pl.ds(e, 1), :]
        return carry

    lax.fori_loop(0, EB, scatter, 0)


def _edge_call(src, dst, edge_feats, we, tk, tq, tv):
    return pl.pallas_call(
        _edge_body,
        grid=(NB,),
        in_specs=[
            pl.BlockSpec((EB,), lambda i: (i,), memory_space=pltpu.SMEM),
            pl.BlockSpec((EB,), lambda i: (i,), memory_space=pltpu.SMEM),
            pl.BlockSpec((EB, DIN), lambda i: (i, 0)),
            pl.BlockSpec((DIN, HD), lambda i: (0, 0)),
            pl.BlockSpec((N, HD), lambda i: (0, 0)),
            pl.BlockSpec((N, HD), lambda i: (0, 0)),
            pl.BlockSpec((N, HD), lambda i: (0, 0)),
        ],
        out_specs=[
            pl.BlockSpec((EB, HD), lambda i: (i, 0)),
            pl.BlockSpec((N, HD), lambda i: (0, 0)),
            pl.BlockSpec((N, HD), lambda i: (0, 0)),
        ],
        out_shape=[
            jax.ShapeDtypeStruct((E, HD), jnp.float32),
            jax.ShapeDtypeStruct((N, HD), jnp.float32),
            jax.ShapeDtypeStruct((N, HD), jnp.float32),
        ],
        scratch_shapes=[pltpu.VMEM((EB, HD), jnp.float32)] * 4,
    )(src, dst, edge_feats, we, tk, tq, tv)


def _div_body(wv_ref, z_ref, o_ref):
    o_ref[...] = wv_ref[...] / (z_ref[...] + 1e-6)


def _div_call(wv, z):
    bn = 2000
    return pl.pallas_call(
        _div_body,
        grid=(N // bn,),
        in_specs=[
            pl.BlockSpec((bn, HD), lambda i: (i, 0)),
            pl.BlockSpec((bn, HD), lambda i: (i, 0)),
        ],
        out_specs=pl.BlockSpec((bn, HD), lambda i: (i, 0)),
        out_shape=jax.ShapeDtypeStruct((N, HD), jnp.float32),
    )(wv, z)


def kernel(node_feats, edge_feats, edge_index, Qw, Kw, Vw, Ew):
    tk, tq, tv = _tab_call(node_feats, (Kw * 0.25).T, Qw.T, Vw.T)
    e_out, wv, z = _edge_call(edge_index[0], edge_index[1], edge_feats,
                              Ew.T, tk, tq, tv)
    h_out = _div_call(wv, z)
    return h_out.reshape(N, H, D), e_out.reshape(E, H, D)
